# Initial kernel scaffold; baseline (speedup 1.0000x reference)
import jax
import jax.numpy as jnp
from jax.experimental import pallas as pl


def kernel(x, split, W, b):
    raise NotImplementedError("write your pallas kernel here")


# speedup vs baseline: 3.2798x; 3.2798x over previous
import functools

import jax
import jax.numpy as jnp
from jax import lax
from jax.experimental import pallas as pl
from jax.experimental.pallas import tpu as pltpu
from jax.experimental.pallas import tpu_sc as plsc

E = 8
N = 8192
D = 1024

BLK = 256
NB = N // BLK + E
NPAD = NB * BLK

NC = 2
NS = 16
NW = NC * NS
TOK_W = N // NW
CH = 64


def _permute_body(src_hbm, idx_hbm, out_hbm, idx_v, rows_v, sem, *, gather):
    wid = lax.axis_index("s") * NC + lax.axis_index("c")
    base = wid * TOK_W
    for c in range(TOK_W // CH):
        off = base + c * CH
        pltpu.sync_copy(idx_hbm.at[pl.ds(off, CH)], idx_v)
        if gather:
            pltpu.async_copy(src_hbm.at[idx_v], rows_v, sem).wait()
            pltpu.sync_copy(rows_v, out_hbm.at[pl.ds(off, CH)])
        else:
            pltpu.sync_copy(src_hbm.at[pl.ds(off, CH)], rows_v)
            pltpu.async_copy(rows_v, out_hbm.at[idx_v], sem).wait()


def _make_permute(out_rows, gather):
    mesh = plsc.VectorSubcoreMesh(
        core_axis_name="c", subcore_axis_name="s",
        num_cores=NC, num_subcores=NS)
    return pl.kernel(
        functools.partial(_permute_body, gather=gather),
        out_type=jax.ShapeDtypeStruct((out_rows, D), jnp.float32),
        mesh=mesh,
        scratch_types=[
            pltpu.VMEM((CH,), jnp.int32),
            pltpu.VMEM((CH, D), jnp.float32),
            pltpu.SemaphoreType.DMA,
        ],
    )


_scatter_rows = _make_permute(NPAD, gather=False)
_gather_rows = _make_permute(N, gather=True)


def _mm_body(be_ref, x_ref, w_ref, b_ref, o_ref):
    acc = lax.dot_general(
        x_ref[...], w_ref[0],
        dimension_numbers=(((1,), (1,)), ((), ())),
        preferred_element_type=jnp.float32)
    o_ref[...] = acc + b_ref[0]


_grouped_mm = pl.pallas_call(
    _mm_body,
    grid_spec=pltpu.PrefetchScalarGridSpec(
        num_scalar_prefetch=1,
        grid=(NB,),
        in_specs=[
            pl.BlockSpec((BLK, D), lambda i, be: (i, 0)),
            pl.BlockSpec((1, D, D), lambda i, be: (be[i], 0, 0)),
            pl.BlockSpec((1, 1, D), lambda i, be: (be[i], 0, 0)),
        ],
        out_specs=pl.BlockSpec((BLK, D), lambda i, be: (i, 0)),
    ),
    out_shape=jax.ShapeDtypeStruct((NPAD, D), jnp.float32),
)


def kernel(x, split, W, b):
    split = split.astype(jnp.int32)
    onehot = (split[:, None] == jnp.arange(E, dtype=jnp.int32)[None, :])
    incl = jnp.cumsum(onehot.astype(jnp.int32), axis=0)
    counts = incl[-1]
    rank = jnp.take_along_axis(incl, split[:, None], axis=1)[:, 0] - 1
    padded = ((counts + BLK - 1) // BLK) * BLK
    bounds = jnp.cumsum(padded)
    pad_off = bounds - padded
    dst = pad_off[split] + rank
    tstart = jnp.arange(NB, dtype=jnp.int32) * BLK
    block_expert = jnp.minimum(
        jnp.searchsorted(bounds, tstart, side="right"), E - 1).astype(jnp.int32)

    xs = _scatter_rows(x, dst)
    ys = _grouped_mm(block_expert, xs, W, b[:, None, :])
    y = _gather_rows(ys, dst)
    return y
